# Initial kernel scaffold; baseline (speedup 1.0000x reference)
import functools

import jax
import jax.numpy as jnp
from jax import lax
from jax.experimental import pallas as pl
from jax.experimental.pallas import tpu as pltpu
from jax.experimental.pallas import tpu_sc as plsc

N = 10000
D = 128
E = 320000

NC = 2
NS = 16
NW = NC * NS

CHUNK = 128
ACC = 10240
STRIPE = ACC // NS
NCH = 80
EP = NW * NCH * CHUNK

MH = 120000
HCH = 30
MHP = NW * HCH * CHUNK
HPT = HCH * CHUNK
HB = 3840
RB = 2000

_mesh = plsc.VectorSubcoreMesh(core_axis_name="c", subcore_axis_name="s")
_f32 = jnp.float32


@functools.partial(
    pl.kernel,
    out_type=jax.ShapeDtypeStruct((NC, ACC, 16), _f32),
    mesh=_mesh,
    scratch_types=[
        pltpu.VMEM((NCH, CHUNK), jnp.int32),
        pltpu.VMEM((CHUNK, 16), _f32),
        pltpu.VMEM_SHARED((ACC, 16), _f32),
    ],
)
def _sc_deg(cols_hbm, ones_hbm, zrow_hbm, out_hbm, colbuf, onesbuf, acc):
    c = lax.axis_index("c")
    s = lax.axis_index("s")
    gw = c * NS + s
    pltpu.sync_copy(cols_hbm.at[gw], colbuf)
    pltpu.sync_copy(ones_hbm, onesbuf)
    pltpu.sync_copy(zrow_hbm, acc.at[pl.ds(s * STRIPE, STRIPE)])
    plsc.subcore_barrier()

    def step(j, carry):
        pltpu.sync_copy(onesbuf, acc.at[colbuf.at[j]], add=True)
        return carry

    lax.fori_loop(0, NCH, step, 0)
    plsc.subcore_barrier()
    pltpu.sync_copy(acc.at[pl.ds(s * STRIPE, STRIPE)],
                    out_hbm.at[c, pl.ds(s * STRIPE, STRIPE)])


@functools.partial(
    pl.kernel,
    out_type=jax.ShapeDtypeStruct((NC, ACC, D), _f32),
    mesh=_mesh,
    scratch_types=[
        pltpu.VMEM((NCH, CHUNK), jnp.int32),
        pltpu.VMEM((NCH, CHUNK), jnp.int32),
        pltpu.VMEM((CHUNK, D), _f32),
        pltpu.VMEM_SHARED((ACC, D), _f32),
    ],
)
def _sc_scatter(u_hbm, rows_hbm, cols_hbm, zrow_hbm, out_hbm,
                rowbuf, colbuf, gbuf, acc):
    c = lax.axis_index("c")
    s = lax.axis_index("s")
    gw = c * NS + s
    pltpu.sync_copy(rows_hbm.at[gw], rowbuf)
    pltpu.sync_copy(cols_hbm.at[gw], colbuf)
    pltpu.sync_copy(zrow_hbm, acc.at[pl.ds(s * STRIPE, STRIPE)])
    plsc.subcore_barrier()

    def step(j, carry):
        pltpu.sync_copy(u_hbm.at[rowbuf.at[j]], gbuf)
        pltpu.sync_copy(gbuf, acc.at[colbuf.at[j]], add=True)
        return carry

    lax.fori_loop(0, NCH, step, 0)
    plsc.subcore_barrier()
    pltpu.sync_copy(acc.at[pl.ds(s * STRIPE, STRIPE)],
                    out_hbm.at[c, pl.ds(s * STRIPE, STRIPE)])


@functools.partial(
    pl.kernel,
    out_type=(jax.ShapeDtypeStruct((MHP, D), _f32),
              jax.ShapeDtypeStruct((MHP, D), _f32)),
    mesh=_mesh,
    scratch_types=[
        pltpu.VMEM((HCH, CHUNK), jnp.int32),
        pltpu.VMEM((HCH, CHUNK), jnp.int32),
        pltpu.VMEM((CHUNK, D), _f32),
        pltpu.VMEM((CHUNK, D), _f32),
    ],
)
def _sc_head_gather(h_hbm, sidx_hbm, didx_hbm, zs_hbm, zd_hbm,
                    sbuf, dbuf, gs, gd):
    c = lax.axis_index("c")
    s = lax.axis_index("s")
    gw = c * NS + s
    pltpu.sync_copy(sidx_hbm.at[gw], sbuf)
    pltpu.sync_copy(didx_hbm.at[gw], dbuf)

    def step(j, carry):
        base = gw * HPT + j * CHUNK
        pltpu.sync_copy(h_hbm.at[sbuf.at[j]], gs)
        pltpu.sync_copy(gs, zs_hbm.at[pl.ds(base, CHUNK)])
        pltpu.sync_copy(h_hbm.at[dbuf.at[j]], gd)
        pltpu.sync_copy(gd, zd_hbm.at[pl.ds(base, CHUNK)])
        return carry

    lax.fori_loop(0, HCH, step, 0)


def _dinv(dp0, dp1):
    deg = dp0[:, :1] + dp1[:, :1] + 1.0
    return lax.rsqrt(deg)


def _tc_first_body(x_ref, w_ref, dp0_ref, dp1_ref, o_ref):
    dinv = _dinv(dp0_ref[...], dp1_ref[...])
    xw = jnp.dot(x_ref[...], w_ref[...], preferred_element_type=_f32)
    o_ref[...] = xw * dinv


def _tc_mid_body(s0_ref, s1_ref, u_ref, dp0_ref, dp1_ref, b_ref, w_ref, o_ref):
    dinv = _dinv(dp0_ref[...], dp1_ref[...])
    h = dinv * (s0_ref[...] + s1_ref[...] + u_ref[...]) + b_ref[...]
    h = jnp.maximum(h, 0.0)
    o_ref[...] = jnp.dot(h, w_ref[...], preferred_element_type=_f32) * dinv


def _tc_last_body(s0_ref, s1_ref, u_ref, dp0_ref, dp1_ref, b_ref, o_ref):
    dinv = _dinv(dp0_ref[...], dp1_ref[...])
    o_ref[...] = dinv * (s0_ref[...] + s1_ref[...] + u_ref[...]) + b_ref[...]


def _tc_head_body(zs_ref, zd_ref, wa_ref, wb_ref, b1_ref, w2_ref, b2_ref,
                  o_ref):
    t = (jnp.dot(zs_ref[...], wa_ref[...], preferred_element_type=_f32)
         + jnp.dot(zd_ref[...], wb_ref[...], preferred_element_type=_f32)
         + b1_ref[...])
    t = jnp.maximum(t, 0.0)
    o_ref[...] = jnp.sum(t * w2_ref[...], axis=1) + b2_ref[0]


_row_spec = pl.BlockSpec((RB, D), lambda i: (i, 0))
_dp_spec = pl.BlockSpec((RB, 16), lambda i: (i, 0))
_w_spec = pl.BlockSpec((D, D), lambda i: (0, 0))
_b_spec = pl.BlockSpec((1, D), lambda i: (0, 0))

_tc_first = pl.pallas_call(
    _tc_first_body,
    grid=(N // RB,),
    in_specs=[_row_spec, _w_spec, _dp_spec, _dp_spec],
    out_specs=_row_spec,
    out_shape=jax.ShapeDtypeStruct((N, D), _f32),
)

_tc_mid = pl.pallas_call(
    _tc_mid_body,
    grid=(N // RB,),
    in_specs=[_row_spec, _row_spec, _row_spec, _dp_spec, _dp_spec, _b_spec,
              _w_spec],
    out_specs=_row_spec,
    out_shape=jax.ShapeDtypeStruct((N, D), _f32),
)

_tc_last = pl.pallas_call(
    _tc_last_body,
    grid=(N // RB,),
    in_specs=[_row_spec, _row_spec, _row_spec, _dp_spec, _dp_spec, _b_spec],
    out_specs=_row_spec,
    out_shape=jax.ShapeDtypeStruct((N, D), _f32),
)

_hrow_spec = pl.BlockSpec((HB, D), lambda i: (i, 0))

_tc_head = pl.pallas_call(
    _tc_head_body,
    grid=(MHP // HB,),
    in_specs=[_hrow_spec, _hrow_spec, _w_spec, _w_spec, _b_spec, _b_spec,
              pl.BlockSpec(memory_space=pltpu.SMEM)],
    out_specs=pl.BlockSpec((HB,), lambda i: (i,)),
    out_shape=jax.ShapeDtypeStruct((MHP,), _f32),
)


def kernel(x, edge_index, pos_edges_train, neg_edges_train, pos_edges_test,
           neg_edges_test, W0, b0, W1, b1, W2, b2, W3, b3, Wl1, bl1, Wl2,
           bl2):
    i32 = jnp.int32
    row = edge_index[0].astype(i32)
    col = edge_index[1].astype(i32)
    rows_p = jnp.concatenate(
        [row, jnp.zeros((EP - E,), i32)]).reshape(NW, NCH, CHUNK)
    cols_p = jnp.concatenate(
        [col, jnp.full((EP - E,), N, i32)]).reshape(NW, NCH, CHUNK)

    ones16 = jnp.ones((CHUNK, 16), _f32)
    zrow16 = jnp.zeros((STRIPE, 16), _f32)
    zrowD = jnp.zeros((STRIPE, D), _f32)

    degp = _sc_deg(cols_p, ones16, zrow16)
    dp0 = degp[0, :N, :16]
    dp1 = degp[1, :N, :16]

    b0r = b0.reshape(1, D)
    b1r = b1.reshape(1, D)
    b2r = b2.reshape(1, D)
    b3r = b3.reshape(1, D)

    u = _tc_first(x, W0, dp0, dp1)
    for bias, w_next in ((b0r, W1), (b1r, W2), (b2r, W3)):
        sp = _sc_scatter(u, rows_p, cols_p, zrowD)
        u = _tc_mid(sp[0, :N], sp[1, :N], u, dp0, dp1, bias, w_next)
    sp = _sc_scatter(u, rows_p, cols_p, zrowD)
    h = _tc_last(sp[0, :N], sp[1, :N], u, dp0, dp1, b3r)

    sidx = jnp.concatenate(
        [pos_edges_train[0], neg_edges_train[0], pos_edges_test[0],
         neg_edges_test[0], jnp.zeros((MHP - MH,), i32)]).astype(i32)
    didx = jnp.concatenate(
        [pos_edges_train[1], neg_edges_train[1], pos_edges_test[1],
         neg_edges_test[1], jnp.zeros((MHP - MH,), i32)]).astype(i32)
    sidx = sidx.reshape(NW, HCH, CHUNK)
    didx = didx.reshape(NW, HCH, CHUNK)

    zs, zd = _sc_head_gather(h, sidx, didx)

    wa = Wl1[:D]
    wb = Wl1[D:]
    z = _tc_head(zs, zd, wa, wb, bl1.reshape(1, D), Wl2.reshape(1, D),
                 bl2.reshape(1))
    z = z[:MH]
    return (z[:100000], z[100000:])


# speedup vs baseline: 5.0677x; 5.0677x over previous
import functools

import jax
import jax.numpy as jnp
from jax import lax
from jax.experimental import pallas as pl
from jax.experimental.pallas import tpu as pltpu
from jax.experimental.pallas import tpu_sc as plsc

N = 10000
D = 128
E = 320000

NC = 2
NS = 16
NW = NC * NS

CHUNK = 128
ACC = 10240
STRIPE = ACC // NS
NCH = 80
EP = NW * NCH * CHUNK

MH = 120000
HCH = 30
MHP = NW * HCH * CHUNK
HPT = HCH * CHUNK
HB = 4096
RB = 2000

_mesh = plsc.VectorSubcoreMesh(core_axis_name="c", subcore_axis_name="s")
_f32 = jnp.float32


@functools.partial(
    pl.kernel,
    out_type=jax.ShapeDtypeStruct((NC, ACC, D), _f32),
    mesh=_mesh,
    scratch_types=[
        pltpu.VMEM((NCH, CHUNK), jnp.int32),
        pltpu.VMEM((CHUNK, D), _f32),
        pltpu.VMEM_SHARED((ACC, D), _f32),
    ],
)
def _sc_deg(cols_hbm, ones_hbm, zrow_hbm, out_hbm, colbuf, onesbuf, acc):
    c = lax.axis_index("c")
    s = lax.axis_index("s")
    gw = c * NS + s
    pltpu.sync_copy(cols_hbm.at[gw], colbuf)
    pltpu.sync_copy(ones_hbm, onesbuf)
    pltpu.sync_copy(zrow_hbm, acc.at[pl.ds(s * STRIPE, STRIPE)])
    plsc.subcore_barrier()

    def step(j, carry):
        pltpu.sync_copy(onesbuf, acc.at[colbuf.at[j]], add=True)
        return carry

    lax.fori_loop(0, NCH, step, 0)
    plsc.subcore_barrier()
    pltpu.sync_copy(acc.at[pl.ds(s * STRIPE, STRIPE)],
                    out_hbm.at[c, pl.ds(s * STRIPE, STRIPE)])


@functools.partial(
    pl.kernel,
    out_type=jax.ShapeDtypeStruct((NC, ACC, D), _f32),
    mesh=_mesh,
    scratch_types=[
        pltpu.VMEM((NCH, CHUNK), jnp.int32),
        pltpu.VMEM((NCH, CHUNK), jnp.int32),
        pltpu.VMEM((CHUNK, D), _f32),
        pltpu.VMEM_SHARED((ACC, D), _f32),
    ],
)
def _sc_scatter(u_hbm, rows_hbm, cols_hbm, zrow_hbm, out_hbm,
                rowbuf, colbuf, gbuf, acc):
    c = lax.axis_index("c")
    s = lax.axis_index("s")
    gw = c * NS + s
    pltpu.sync_copy(rows_hbm.at[gw], rowbuf)
    pltpu.sync_copy(cols_hbm.at[gw], colbuf)
    pltpu.sync_copy(zrow_hbm, acc.at[pl.ds(s * STRIPE, STRIPE)])
    plsc.subcore_barrier()

    def step(j, carry):
        pltpu.sync_copy(u_hbm.at[rowbuf.at[j]], gbuf)
        pltpu.sync_copy(gbuf, acc.at[colbuf.at[j]], add=True)
        return carry

    lax.fori_loop(0, NCH, step, 0)
    plsc.subcore_barrier()
    pltpu.sync_copy(acc.at[pl.ds(s * STRIPE, STRIPE)],
                    out_hbm.at[c, pl.ds(s * STRIPE, STRIPE)])


@functools.partial(
    pl.kernel,
    out_type=(jax.ShapeDtypeStruct((MHP, D), _f32),
              jax.ShapeDtypeStruct((MHP, D), _f32)),
    mesh=_mesh,
    scratch_types=[
        pltpu.VMEM((HCH, CHUNK), jnp.int32),
        pltpu.VMEM((HCH, CHUNK), jnp.int32),
        pltpu.VMEM((CHUNK, D), _f32),
        pltpu.VMEM((CHUNK, D), _f32),
    ],
)
def _sc_head_gather(h_hbm, sidx_hbm, didx_hbm, zs_hbm, zd_hbm,
                    sbuf, dbuf, gs, gd):
    c = lax.axis_index("c")
    s = lax.axis_index("s")
    gw = c * NS + s
    pltpu.sync_copy(sidx_hbm.at[gw], sbuf)
    pltpu.sync_copy(didx_hbm.at[gw], dbuf)

    def step(j, carry):
        base = gw * HPT + j * CHUNK
        pltpu.sync_copy(h_hbm.at[sbuf.at[j]], gs)
        pltpu.sync_copy(gs, zs_hbm.at[pl.ds(base, CHUNK)])
        pltpu.sync_copy(h_hbm.at[dbuf.at[j]], gd)
        pltpu.sync_copy(gd, zd_hbm.at[pl.ds(base, CHUNK)])
        return carry

    lax.fori_loop(0, HCH, step, 0)


def _dinv(dp0, dp1):
    deg = dp0[:, :1] + dp1[:, :1] + 1.0
    return lax.rsqrt(deg)


def _tc_first_body(x_ref, w_ref, dp0_ref, dp1_ref, o_ref):
    dinv = _dinv(dp0_ref[...], dp1_ref[...])
    xw = jnp.dot(x_ref[...], w_ref[...], preferred_element_type=_f32)
    o_ref[...] = xw * dinv


def _tc_mid_body(s0_ref, s1_ref, u_ref, dp0_ref, dp1_ref, b_ref, w_ref, o_ref):
    dinv = _dinv(dp0_ref[...], dp1_ref[...])
    h = dinv * (s0_ref[...] + s1_ref[...] + u_ref[...]) + b_ref[...]
    h = jnp.maximum(h, 0.0)
    o_ref[...] = jnp.dot(h, w_ref[...], preferred_element_type=_f32) * dinv


def _tc_last_body(s0_ref, s1_ref, u_ref, dp0_ref, dp1_ref, b_ref, o_ref):
    dinv = _dinv(dp0_ref[...], dp1_ref[...])
    o_ref[...] = dinv * (s0_ref[...] + s1_ref[...] + u_ref[...]) + b_ref[...]


def _tc_head_body(zs_ref, zd_ref, wa_ref, wb_ref, b1_ref, w2_ref, b2_ref,
                  o_ref):
    t = (jnp.dot(zs_ref[...], wa_ref[...], preferred_element_type=_f32)
         + jnp.dot(zd_ref[...], wb_ref[...], preferred_element_type=_f32)
         + b1_ref[...])
    t = jnp.maximum(t, 0.0)
    o_ref[...] = jnp.sum(t * w2_ref[...], axis=1) + b2_ref[0]


_row_spec = pl.BlockSpec((RB, D), lambda i: (i, 0))
_dp_spec = pl.BlockSpec((RB, 16), lambda i: (i, 0))
_w_spec = pl.BlockSpec((D, D), lambda i: (0, 0))
_b_spec = pl.BlockSpec((1, D), lambda i: (0, 0))

_tc_first = pl.pallas_call(
    _tc_first_body,
    grid=(N // RB,),
    in_specs=[_row_spec, _w_spec, _dp_spec, _dp_spec],
    out_specs=_row_spec,
    out_shape=jax.ShapeDtypeStruct((N, D), _f32),
)

_tc_mid = pl.pallas_call(
    _tc_mid_body,
    grid=(N // RB,),
    in_specs=[_row_spec, _row_spec, _row_spec, _dp_spec, _dp_spec, _b_spec,
              _w_spec],
    out_specs=_row_spec,
    out_shape=jax.ShapeDtypeStruct((N, D), _f32),
)

_tc_last = pl.pallas_call(
    _tc_last_body,
    grid=(N // RB,),
    in_specs=[_row_spec, _row_spec, _row_spec, _dp_spec, _dp_spec, _b_spec],
    out_specs=_row_spec,
    out_shape=jax.ShapeDtypeStruct((N, D), _f32),
)

_hrow_spec = pl.BlockSpec((HB, D), lambda i: (i, 0))

_tc_head = pl.pallas_call(
    _tc_head_body,
    grid=(MHP // HB,),
    in_specs=[_hrow_spec, _hrow_spec, _w_spec, _w_spec, _b_spec, _b_spec,
              pl.BlockSpec(memory_space=pltpu.SMEM)],
    out_specs=pl.BlockSpec((HB,), lambda i: (i,)),
    out_shape=jax.ShapeDtypeStruct((MHP,), _f32),
)


def kernel(x, edge_index, pos_edges_train, neg_edges_train, pos_edges_test,
           neg_edges_test, W0, b0, W1, b1, W2, b2, W3, b3, Wl1, bl1, Wl2,
           bl2):
    i32 = jnp.int32
    row = edge_index[0].astype(i32)
    col = edge_index[1].astype(i32)
    rows_p = jnp.concatenate(
        [row, jnp.zeros((EP - E,), i32)]).reshape(NW, NCH, CHUNK)
    cols_p = jnp.concatenate(
        [col, jnp.full((EP - E,), N, i32)]).reshape(NW, NCH, CHUNK)

    onesD = jnp.ones((CHUNK, D), _f32)
    zrowD = jnp.zeros((STRIPE, D), _f32)

    degp = _sc_deg(cols_p, onesD, zrowD)
    dp0 = degp[0, :N, :16]
    dp1 = degp[1, :N, :16]

    b0r = b0.reshape(1, D)
    b1r = b1.reshape(1, D)
    b2r = b2.reshape(1, D)
    b3r = b3.reshape(1, D)

    u = _tc_first(x, W0, dp0, dp1)
    for bias, w_next in ((b0r, W1), (b1r, W2), (b2r, W3)):
        sp = _sc_scatter(u, rows_p, cols_p, zrowD)
        u = _tc_mid(sp[0, :N], sp[1, :N], u, dp0, dp1, bias, w_next)
    sp = _sc_scatter(u, rows_p, cols_p, zrowD)
    h = _tc_last(sp[0, :N], sp[1, :N], u, dp0, dp1, b3r)

    sidx = jnp.concatenate(
        [pos_edges_train[0], neg_edges_train[0], pos_edges_test[0],
         neg_edges_test[0], jnp.zeros((MHP - MH,), i32)]).astype(i32)
    didx = jnp.concatenate(
        [pos_edges_train[1], neg_edges_train[1], pos_edges_test[1],
         neg_edges_test[1], jnp.zeros((MHP - MH,), i32)]).astype(i32)
    sidx = sidx.reshape(NW, HCH, CHUNK)
    didx = didx.reshape(NW, HCH, CHUNK)

    zs, zd = _sc_head_gather(h, sidx, didx)

    wa = Wl1[:D]
    wb = Wl1[D:]
    z = _tc_head(zs, zd, wa, wb, bl1.reshape(1, D), Wl2.reshape(1, D),
                 bl2.reshape(1))
    z = z[:MH]
    return (z[:100000], z[100000:])


# speedup vs baseline: 12.8290x; 2.5315x over previous
import functools

import jax
import jax.numpy as jnp
from jax import lax
from jax.experimental import pallas as pl
from jax.experimental.pallas import tpu as pltpu
from jax.experimental.pallas import tpu_sc as plsc

N = 10000
D = 128
E = 320000

NC = 2
NS = 16
NW = NC * NS

CHUNK = 128
ACC = 10240
STRIPE = ACC // NS
NCH = 80
EP = NW * NCH * CHUNK

MH = 120000
HCH = 30
MHP = NW * HCH * CHUNK
HPT = HCH * CHUNK
HB = 4096
RB = 2000

_mesh = plsc.VectorSubcoreMesh(core_axis_name="c", subcore_axis_name="s")
_f32 = jnp.float32


NBUF = 4


@functools.partial(
    pl.kernel,
    out_type=jax.ShapeDtypeStruct((NC, ACC, D), _f32),
    mesh=_mesh,
    scratch_types=[
        pltpu.VMEM((NCH, CHUNK), jnp.int32),
        pltpu.VMEM((CHUNK, D), _f32),
        pltpu.VMEM_SHARED((ACC, D), _f32),
        pltpu.SemaphoreType.DMA,
    ],
)
def _sc_deg(cols_hbm, ones_hbm, zrow_hbm, out_hbm, colbuf, onesbuf, acc, sem):
    c = lax.axis_index("c")
    s = lax.axis_index("s")
    gw = c * NS + s
    pltpu.sync_copy(cols_hbm.at[gw], colbuf)
    pltpu.sync_copy(ones_hbm, onesbuf)
    pltpu.sync_copy(zrow_hbm, acc.at[pl.ds(s * STRIPE, STRIPE)])
    plsc.subcore_barrier()

    K = 8

    def group(g, carry):
        for t in range(K):
            pltpu.async_copy(onesbuf, acc.at[colbuf.at[g * K + t]], sem,
                             add=True)
        for t in range(K):
            pltpu.make_async_copy(onesbuf, acc.at[colbuf.at[g * K + t]],
                                  sem).wait()
        return carry

    lax.fori_loop(0, NCH // K, group, 0)
    plsc.subcore_barrier()
    pltpu.sync_copy(acc.at[pl.ds(s * STRIPE, STRIPE)],
                    out_hbm.at[c, pl.ds(s * STRIPE, STRIPE)])


SGK = 4


@functools.partial(
    pl.kernel,
    out_type=jax.ShapeDtypeStruct((NC, ACC, D), _f32),
    mesh=_mesh,
    scratch_types=[
        pltpu.VMEM((NCH, CHUNK), jnp.int32),
        pltpu.VMEM((NCH, CHUNK), jnp.int32),
        pltpu.VMEM((CHUNK, D), _f32),
        pltpu.VMEM_SHARED((ACC, D), _f32),
    ],
)
def _sc_scatter(u_hbm, rows_hbm, cols_hbm, zrow_hbm, out_hbm,
                rowbuf, colbuf, gbuf, acc):
    c = lax.axis_index("c")
    s = lax.axis_index("s")
    gw = c * NS + s
    pltpu.sync_copy(rows_hbm.at[gw], rowbuf)
    pltpu.sync_copy(cols_hbm.at[gw], colbuf)
    pltpu.sync_copy(zrow_hbm, acc.at[pl.ds(s * STRIPE, STRIPE)])
    plsc.subcore_barrier()

    def step(j, carry):
        pltpu.sync_copy(u_hbm.at[rowbuf.at[j]], gbuf)
        pltpu.sync_copy(gbuf, acc.at[colbuf.at[j]], add=True)
        return carry

    lax.fori_loop(0, NCH, step, 0)
    plsc.subcore_barrier()
    pltpu.sync_copy(acc.at[pl.ds(s * STRIPE, STRIPE)],
                    out_hbm.at[c, pl.ds(s * STRIPE, STRIPE)])


@functools.partial(
    pl.kernel,
    out_type=(jax.ShapeDtypeStruct((MHP, D), _f32),
              jax.ShapeDtypeStruct((MHP, D), _f32)),
    mesh=_mesh,
    scratch_types=[
        pltpu.VMEM((HCH, CHUNK), jnp.int32),
        pltpu.VMEM((HCH, CHUNK), jnp.int32),
        pltpu.VMEM((CHUNK, D), _f32),
        pltpu.VMEM((CHUNK, D), _f32),
    ],
)
def _sc_head_gather(h_hbm, sidx_hbm, didx_hbm, zs_hbm, zd_hbm,
                    sbuf, dbuf, gs, gd):
    c = lax.axis_index("c")
    s = lax.axis_index("s")
    gw = c * NS + s
    pltpu.sync_copy(sidx_hbm.at[gw], sbuf)
    pltpu.sync_copy(didx_hbm.at[gw], dbuf)

    def step(j, carry):
        base = gw * HPT + j * CHUNK
        pltpu.sync_copy(h_hbm.at[sbuf.at[j]], gs)
        pltpu.sync_copy(gs, zs_hbm.at[pl.ds(base, CHUNK)])
        pltpu.sync_copy(h_hbm.at[dbuf.at[j]], gd)
        pltpu.sync_copy(gd, zd_hbm.at[pl.ds(base, CHUNK)])
        return carry

    lax.fori_loop(0, HCH, step, 0)


def _dinv(dp0, dp1):
    deg = dp0[:, :1] + dp1[:, :1] + 1.0
    return lax.rsqrt(deg)


def _tc_first_body(x_ref, w_ref, dp0_ref, dp1_ref, o_ref):
    dinv = _dinv(dp0_ref[...], dp1_ref[...])
    xw = jnp.dot(x_ref[...], w_ref[...], preferred_element_type=_f32)
    o_ref[...] = xw * dinv


def _tc_mid_body(s0_ref, s1_ref, u_ref, dp0_ref, dp1_ref, b_ref, w_ref, o_ref):
    dinv = _dinv(dp0_ref[...], dp1_ref[...])
    h = dinv * (s0_ref[...] + s1_ref[...] + u_ref[...]) + b_ref[...]
    h = jnp.maximum(h, 0.0)
    o_ref[...] = jnp.dot(h, w_ref[...], preferred_element_type=_f32) * dinv


def _tc_last_body(s0_ref, s1_ref, u_ref, dp0_ref, dp1_ref, b_ref, o_ref):
    dinv = _dinv(dp0_ref[...], dp1_ref[...])
    o_ref[...] = dinv * (s0_ref[...] + s1_ref[...] + u_ref[...]) + b_ref[...]


def _tc_head_body(zs_ref, zd_ref, wa_ref, wb_ref, b1_ref, w2_ref, b2_ref,
                  o_ref):
    t = (jnp.dot(zs_ref[...], wa_ref[...], preferred_element_type=_f32)
         + jnp.dot(zd_ref[...], wb_ref[...], preferred_element_type=_f32)
         + b1_ref[...])
    t = jnp.maximum(t, 0.0)
    o_ref[...] = jnp.sum(t * w2_ref[...], axis=1) + b2_ref[0]


_row_spec = pl.BlockSpec((RB, D), lambda i: (i, 0))
_dp_spec = pl.BlockSpec((RB, 16), lambda i: (i, 0))
_w_spec = pl.BlockSpec((D, D), lambda i: (0, 0))
_b_spec = pl.BlockSpec((1, D), lambda i: (0, 0))

_tc_first = pl.pallas_call(
    _tc_first_body,
    grid=(N // RB,),
    in_specs=[_row_spec, _w_spec, _dp_spec, _dp_spec],
    out_specs=_row_spec,
    out_shape=jax.ShapeDtypeStruct((N, D), _f32),
)

_tc_mid = pl.pallas_call(
    _tc_mid_body,
    grid=(N // RB,),
    in_specs=[_row_spec, _row_spec, _row_spec, _dp_spec, _dp_spec, _b_spec,
              _w_spec],
    out_specs=_row_spec,
    out_shape=jax.ShapeDtypeStruct((N, D), _f32),
)

_tc_last = pl.pallas_call(
    _tc_last_body,
    grid=(N // RB,),
    in_specs=[_row_spec, _row_spec, _row_spec, _dp_spec, _dp_spec, _b_spec],
    out_specs=_row_spec,
    out_shape=jax.ShapeDtypeStruct((N, D), _f32),
)

_hrow_spec = pl.BlockSpec((HB, D), lambda i: (i, 0))

_tc_head = pl.pallas_call(
    _tc_head_body,
    grid=(MHP // HB,),
    in_specs=[_hrow_spec, _hrow_spec, _w_spec, _w_spec, _b_spec, _b_spec,
              pl.BlockSpec(memory_space=pltpu.SMEM)],
    out_specs=pl.BlockSpec((HB,), lambda i: (i,)),
    out_shape=jax.ShapeDtypeStruct((MHP,), _f32),
)


def kernel(x, edge_index, pos_edges_train, neg_edges_train, pos_edges_test,
           neg_edges_test, W0, b0, W1, b1, W2, b2, W3, b3, Wl1, bl1, Wl2,
           bl2):
    i32 = jnp.int32
    row = edge_index[0].astype(i32)
    col = edge_index[1].astype(i32)
    pad = EP - E
    prow = (jnp.arange(pad, dtype=i32) * 13) % N
    pcol = N + (jnp.arange(pad, dtype=i32) % (ACC - N))
    rows_p = jnp.concatenate([row, prow]).reshape(NW, NCH, CHUNK)
    cols_p = jnp.concatenate([col, pcol]).reshape(NW, NCH, CHUNK)

    onesD = jnp.ones((CHUNK, D), _f32)
    zrowD = jnp.zeros((STRIPE, D), _f32)

    degp = _sc_deg(cols_p, onesD, zrowD)
    dp0 = degp[0, :N, :16]
    dp1 = degp[1, :N, :16]

    b0r = b0.reshape(1, D)
    b1r = b1.reshape(1, D)
    b2r = b2.reshape(1, D)
    b3r = b3.reshape(1, D)

    u = _tc_first(x, W0, dp0, dp1)
    for bias, w_next in ((b0r, W1), (b1r, W2), (b2r, W3)):
        sp = _sc_scatter(u, rows_p, cols_p, zrowD)
        u = _tc_mid(sp[0, :N], sp[1, :N], u, dp0, dp1, bias, w_next)
    sp = _sc_scatter(u, rows_p, cols_p, zrowD)
    h = _tc_last(sp[0, :N], sp[1, :N], u, dp0, dp1, b3r)

    hpad = (jnp.arange(MHP - MH, dtype=i32) * 13) % N
    sidx = jnp.concatenate(
        [pos_edges_train[0], neg_edges_train[0], pos_edges_test[0],
         neg_edges_test[0], hpad]).astype(i32)
    didx = jnp.concatenate(
        [pos_edges_train[1], neg_edges_train[1], pos_edges_test[1],
         neg_edges_test[1], hpad]).astype(i32)
    sidx = sidx.reshape(NW, HCH, CHUNK)
    didx = didx.reshape(NW, HCH, CHUNK)

    zs, zd = _sc_head_gather(h, sidx, didx)

    wa = Wl1[:D]
    wb = Wl1[D:]
    z = _tc_head(zs, zd, wa, wb, bl1.reshape(1, D), Wl2.reshape(1, D),
                 bl2.reshape(1))
    z = z[:MH]
    return (z[:100000], z[100000:])


# speedup vs baseline: 13.8720x; 1.0813x over previous
import functools

import jax
import jax.numpy as jnp
from jax import lax
from jax.experimental import pallas as pl
from jax.experimental.pallas import tpu as pltpu
from jax.experimental.pallas import tpu_sc as plsc

N = 10000
D = 128
E = 320000

NC = 2
NS = 16
NW = NC * NS

CHUNK = 128
ACC = 10240
STRIPE = ACC // NS
NCH = 80
EP = NW * NCH * CHUNK

MH = 120000
HCH = 30
MHP = NW * HCH * CHUNK
HPT = HCH * CHUNK
HB = 4096
RB = 2000

_mesh = plsc.VectorSubcoreMesh(core_axis_name="c", subcore_axis_name="s")
_f32 = jnp.float32


NBUF = 4


@functools.partial(
    pl.kernel,
    out_type=jax.ShapeDtypeStruct((NC, ACC, D), _f32),
    mesh=_mesh,
    scratch_types=[
        pltpu.VMEM((NCH, CHUNK), jnp.int32),
        pltpu.VMEM((CHUNK, D), _f32),
        pltpu.VMEM_SHARED((ACC, D), _f32),
        pltpu.SemaphoreType.DMA,
    ],
)
def _sc_deg(cols_hbm, ones_hbm, zrow_hbm, out_hbm, colbuf, onesbuf, acc, sem):
    c = lax.axis_index("c")
    s = lax.axis_index("s")
    gw = c * NS + s
    pltpu.sync_copy(cols_hbm.at[gw], colbuf)
    pltpu.sync_copy(ones_hbm, onesbuf)
    pltpu.sync_copy(zrow_hbm, acc.at[pl.ds(s * STRIPE, STRIPE)])
    plsc.subcore_barrier()

    K = 8

    def group(g, carry):
        for t in range(K):
            pltpu.async_copy(onesbuf, acc.at[colbuf.at[g * K + t]], sem,
                             add=True)
        for t in range(K):
            pltpu.make_async_copy(onesbuf, acc.at[colbuf.at[g * K + t]],
                                  sem).wait()
        return carry

    lax.fori_loop(0, NCH // K, group, 0)
    plsc.subcore_barrier()
    pltpu.sync_copy(acc.at[pl.ds(s * STRIPE, STRIPE)],
                    out_hbm.at[c, pl.ds(s * STRIPE, STRIPE)])


SGK = 4


@functools.partial(
    pl.kernel,
    out_type=jax.ShapeDtypeStruct((NC, ACC, D), _f32),
    mesh=_mesh,
    scratch_types=[
        pltpu.VMEM((NCH, CHUNK), jnp.int32),
        pltpu.VMEM((NCH, CHUNK), jnp.int32),
        pltpu.VMEM((CHUNK, D), _f32),
        pltpu.VMEM_SHARED((ACC, D), _f32),
    ],
)
def _sc_scatter(u_hbm, rows_hbm, cols_hbm, zrow_hbm, out_hbm,
                rowbuf, colbuf, gbuf, acc):
    c = lax.axis_index("c")
    s = lax.axis_index("s")
    gw = c * NS + s
    pltpu.sync_copy(rows_hbm.at[gw], rowbuf)
    pltpu.sync_copy(cols_hbm.at[gw], colbuf)
    pltpu.sync_copy(zrow_hbm, acc.at[pl.ds(s * STRIPE, STRIPE)])
    plsc.subcore_barrier()

    def step(j, carry):
        pltpu.sync_copy(u_hbm.at[rowbuf.at[j]], gbuf)
        pltpu.sync_copy(gbuf, acc.at[colbuf.at[j]], add=True)
        return carry

    lax.fori_loop(0, NCH, step, 0)
    plsc.subcore_barrier()
    pltpu.sync_copy(acc.at[pl.ds(s * STRIPE, STRIPE)],
                    out_hbm.at[c, pl.ds(s * STRIPE, STRIPE)])


@functools.partial(
    pl.kernel,
    out_type=jax.ShapeDtypeStruct((MHP, D), _f32),
    mesh=_mesh,
    scratch_types=[
        pltpu.VMEM((HCH, CHUNK), jnp.int32),
        pltpu.VMEM((HCH, CHUNK), jnp.int32),
        pltpu.VMEM((CHUNK, D), _f32),
    ],
)
def _sc_head_gather(p_hbm, q_hbm, sidx_hbm, didx_hbm, z1_hbm,
                    sbuf, dbuf, buf):
    c = lax.axis_index("c")
    s = lax.axis_index("s")
    gw = c * NS + s
    pltpu.sync_copy(sidx_hbm.at[gw], sbuf)
    pltpu.sync_copy(didx_hbm.at[gw], dbuf)

    def step(j, carry):
        base = gw * HPT + j * CHUNK
        pltpu.sync_copy(p_hbm.at[sbuf.at[j]], buf)
        pltpu.sync_copy(q_hbm.at[dbuf.at[j]], buf, add=True)
        pltpu.sync_copy(buf, z1_hbm.at[pl.ds(base, CHUNK)])
        return carry

    lax.fori_loop(0, HCH, step, 0)


def _dinv(dp0, dp1):
    deg = dp0[:, :1] + dp1[:, :1] + 1.0
    return lax.rsqrt(deg)


def _tc_first_body(x_ref, w_ref, o_ref):
    o_ref[...] = jnp.dot(x_ref[...], w_ref[...], preferred_element_type=_f32)


def _tc_scale_body(xw_ref, dp0_ref, dp1_ref, o_ref):
    o_ref[...] = xw_ref[...] * _dinv(dp0_ref[...], dp1_ref[...])


def _tc_mid_body(sp0_ref, sp1_ref, u_ref, dp0_ref, dp1_ref, b_ref, w_ref,
                 o_ref):
    dinv = _dinv(dp0_ref[...], dp1_ref[...])
    h = dinv * (sp0_ref[0] + sp1_ref[0] + u_ref[...]) + b_ref[...]
    h = jnp.maximum(h, 0.0)
    o_ref[...] = jnp.dot(h, w_ref[...], preferred_element_type=_f32) * dinv


def _tc_last_body(sp0_ref, sp1_ref, u_ref, dp0_ref, dp1_ref, b_ref,
                  wa_ref, wb_ref, p_ref, q_ref):
    dinv = _dinv(dp0_ref[...], dp1_ref[...])
    h = dinv * (sp0_ref[0] + sp1_ref[0] + u_ref[...]) + b_ref[...]
    p_ref[...] = jnp.dot(h, wa_ref[...], preferred_element_type=_f32)
    q_ref[...] = jnp.dot(h, wb_ref[...], preferred_element_type=_f32)


def _tc_head_body(z1_ref, b1_ref, w2_ref, b2_ref, o_ref):
    t = jnp.maximum(z1_ref[...] + b1_ref[...], 0.0)
    o_ref[...] = jnp.sum(t * w2_ref[...], axis=1) + b2_ref[0]


_row_spec = pl.BlockSpec((RB, D), lambda i: (i, 0))
_sp_spec0 = pl.BlockSpec((1, RB, D), lambda i: (0, i, 0))
_sp_spec1 = pl.BlockSpec((1, RB, D), lambda i: (1, i, 0))
_dp_spec = pl.BlockSpec((RB, 16), lambda i: (i, 0))
_w_spec = pl.BlockSpec((D, D), lambda i: (0, 0))
_b_spec = pl.BlockSpec((1, D), lambda i: (0, 0))

_tc_first = pl.pallas_call(
    _tc_first_body,
    grid=(N // RB,),
    in_specs=[_row_spec, _w_spec],
    out_specs=_row_spec,
    out_shape=jax.ShapeDtypeStruct((N, D), _f32),
)

_tc_scale = pl.pallas_call(
    _tc_scale_body,
    grid=(N // RB,),
    in_specs=[_row_spec, _dp_spec, _dp_spec],
    out_specs=_row_spec,
    out_shape=jax.ShapeDtypeStruct((N, D), _f32),
)

_tc_mid = pl.pallas_call(
    _tc_mid_body,
    grid=(N // RB,),
    in_specs=[_sp_spec0, _sp_spec1, _row_spec, _dp_spec, _dp_spec, _b_spec,
              _w_spec],
    out_specs=_row_spec,
    out_shape=jax.ShapeDtypeStruct((N, D), _f32),
)

_tc_last = pl.pallas_call(
    _tc_last_body,
    grid=(N // RB,),
    in_specs=[_sp_spec0, _sp_spec1, _row_spec, _dp_spec, _dp_spec, _b_spec,
              _w_spec, _w_spec],
    out_specs=[_row_spec, _row_spec],
    out_shape=(jax.ShapeDtypeStruct((N, D), _f32),
               jax.ShapeDtypeStruct((N, D), _f32)),
)

_hrow_spec = pl.BlockSpec((HB, D), lambda i: (i, 0))

_tc_head = pl.pallas_call(
    _tc_head_body,
    grid=(MHP // HB,),
    in_specs=[_hrow_spec, _b_spec, _b_spec,
              pl.BlockSpec(memory_space=pltpu.SMEM)],
    out_specs=pl.BlockSpec((HB,), lambda i: (i,)),
    out_shape=jax.ShapeDtypeStruct((MHP,), _f32),
)


def kernel(x, edge_index, pos_edges_train, neg_edges_train, pos_edges_test,
           neg_edges_test, W0, b0, W1, b1, W2, b2, W3, b3, Wl1, bl1, Wl2,
           bl2):
    i32 = jnp.int32
    row = edge_index[0].astype(i32)
    col = edge_index[1].astype(i32)
    pad = EP - E
    prow = (jnp.arange(pad, dtype=i32) * 13) % N
    pcol = N + (jnp.arange(pad, dtype=i32) % (ACC - N))
    rows_p = jnp.concatenate([row, prow]).reshape(NW, NCH, CHUNK)
    cols_p = jnp.concatenate([col, pcol]).reshape(NW, NCH, CHUNK)

    onesD = jnp.ones((CHUNK, D), _f32)
    zrowD = jnp.zeros((STRIPE, D), _f32)

    xw = _tc_first(x, W0)
    degp = _sc_deg(cols_p, onesD, zrowD)
    dp0 = degp[0, :N, :16]
    dp1 = degp[1, :N, :16]
    u = _tc_scale(xw, dp0, dp1)

    b0r = b0.reshape(1, D)
    b1r = b1.reshape(1, D)
    b2r = b2.reshape(1, D)
    b3r = b3.reshape(1, D)

    for bias, w_next in ((b0r, W1), (b1r, W2), (b2r, W3)):
        sp = _sc_scatter(u, rows_p, cols_p, zrowD)
        u = _tc_mid(sp, sp, u, dp0, dp1, bias, w_next)
    sp = _sc_scatter(u, rows_p, cols_p, zrowD)
    p, q = _tc_last(sp, sp, u, dp0, dp1, b3r, Wl1[:D], Wl1[D:])

    hpad = (jnp.arange(MHP - MH, dtype=i32) * 13) % N
    sidx = jnp.concatenate(
        [pos_edges_train[0], neg_edges_train[0], pos_edges_test[0],
         neg_edges_test[0], hpad]).astype(i32)
    didx = jnp.concatenate(
        [pos_edges_train[1], neg_edges_train[1], pos_edges_test[1],
         neg_edges_test[1], hpad]).astype(i32)
    sidx = sidx.reshape(NW, HCH, CHUNK)
    didx = didx.reshape(NW, HCH, CHUNK)

    z1 = _sc_head_gather(p, q, sidx, didx)

    z = _tc_head(z1, bl1.reshape(1, D), Wl2.reshape(1, D), bl2.reshape(1))
    z = z[:MH]
    return (z[:100000], z[100000:])
